# baseline (speedup 1.0000x reference)
import functools

import jax
import jax.numpy as jnp
from jax import lax
from jax.experimental import pallas as pl
from jax.experimental.pallas import tpu as pltpu
from jax.experimental.pallas import tpu_sc as plsc

B = 16384
EMB = 16
NU = 1000000
NC = 2
NS = 16
L = 16
NW = NC * NS
CHW = 1024
NCHF = 976
TAIL0 = NCHF * CHW
TAILN = NU - TAIL0
LCAP = 1024
VCAP = 768
CHCAP = 64
DUMMY = B

_mesh = plsc.VectorSubcoreMesh(core_axis_name="c", subcore_axis_name="s")


@functools.partial(
    pl.kernel,
    out_type=[
        jax.ShapeDtypeStruct((NW * VCAP * EMB,), jnp.float32),
        jax.ShapeDtypeStruct((NW * VCAP * EMB,), jnp.float32),
        jax.ShapeDtypeStruct((NW * LCAP,), jnp.float32),
        jax.ShapeDtypeStruct((NW * LCAP,), jnp.float32),
        jax.ShapeDtypeStruct((NW * LCAP,), jnp.int32),
        jax.ShapeDtypeStruct((NW * LCAP,), jnp.int32),
    ],
    mesh=_mesh,
    compiler_params=pltpu.CompilerParams(needs_layout_passes=False),
    scratch_types=[
        pltpu.VMEM((2, EMB, CHW), jnp.float32),
        pltpu.VMEM((2, EMB, CHW), jnp.float32),
        pltpu.VMEM((CHW,), jnp.float32),
        pltpu.VMEM((CHW,), jnp.float32),
        pltpu.VMEM((CHW,), jnp.float32),
        pltpu.VMEM((CHW,), jnp.float32),
        pltpu.VMEM((2048,), jnp.int32),
        pltpu.VMEM((2048,), jnp.int32),
        pltpu.VMEM((LCAP,), jnp.int32),
        pltpu.VMEM((LCAP,), jnp.int32),
        pltpu.VMEM((LCAP,), jnp.int32),
        pltpu.VMEM((LCAP,), jnp.int32),
        pltpu.VMEM((CHCAP,), jnp.int32),
        pltpu.VMEM((CHCAP,), jnp.int32),
        pltpu.VMEM((VCAP * EMB,), jnp.float32),
        pltpu.VMEM((VCAP * EMB,), jnp.float32),
        pltpu.VMEM((LCAP,), jnp.float32),
        pltpu.VMEM((LCAP,), jnp.float32),
        pltpu.VMEM((TAILN * EMB,), jnp.float32),
        pltpu.VMEM((TAILN * EMB,), jnp.float32),
        pltpu.VMEM((TAILN,), jnp.float32),
        pltpu.VMEM((TAILN,), jnp.float32),
        pltpu.SemaphoreType.DMA,
        pltpu.SemaphoreType.DMA,
        pltpu.SemaphoreType.DMA,
        pltpu.SemaphoreType.DMA,
        pltpu.SemaphoreType.DMA,
        pltpu.SemaphoreType.DMA,
        pltpu.SemaphoreType.DMA,
        pltpu.SemaphoreType.DMA,
        pltpu.SemaphoreType.DMA,
        pltpu.SemaphoreType.DMA,
        pltpu.SemaphoreType.DMA,
        pltpu.SemaphoreType.DMA,
        pltpu.SemaphoreType.DMA,
        pltpu.SemaphoreType.DMA,
    ],
)
def _sc_scan(uT, iT, ub1, ib1, tu, ti, tub, tib, uidx, iidx,
             uvals_o, ivals_o, ubv_o, ibv_o, uslot_o, islot_o,
             uch_v, ich_v, ubc0_v, ubc1_v, ibc0_v, ibc1_v,
             stage0_v, stage1_v,
             uli_v, ulb_v, ili_v, ilb_v, chl_v, chs_v,
             uvv_v, ivv_v, ubv_v, ibv_v, tu_v, ti_v, tub_v, tib_v,
             semu0, semu1, semi0, semi1, semub0, semub1, semib0, semib1,
             sems0, sems1, semtu, semti, semtub, semtib):
    wid = lax.axis_index("s") * NC + lax.axis_index("c")
    iota = lax.iota(jnp.int32, L)

    def fire_chunk(k, buf, bufsems):
        @pl.when((k < NCHF) & (k < 0))
        def _():
            st = pl.multiple_of(k * CHW, CHW)
            su, si, sub, sib = bufsems
            ubc_v = ubc0_v if buf == 0 else ubc1_v
            ibc_v = ibc0_v if buf == 0 else ibc1_v
            pltpu.async_copy(uT.at[:, pl.ds(st, CHW)], uch_v.at[buf], su)
            pltpu.async_copy(iT.at[:, pl.ds(st, CHW)], ich_v.at[buf], si)
            pltpu.async_copy(ub1.at[pl.ds(st, CHW)], ubc_v, sub)
            pltpu.async_copy(ib1.at[pl.ds(st, CHW)], ibc_v, sib)

    def wait_chunk(k, buf, bufsems):
        @pl.when((k < NCHF) & (k < 0))
        def _():
            st = pl.multiple_of(k * CHW, CHW)
            su, si, sub, sib = bufsems
            ubc_v = ubc0_v if buf == 0 else ubc1_v
            ibc_v = ibc0_v if buf == 0 else ibc1_v
            pltpu.make_async_copy(
                uT.at[:, pl.ds(st, CHW)], uch_v.at[buf], su).wait()
            pltpu.make_async_copy(
                iT.at[:, pl.ds(st, CHW)], ich_v.at[buf], si).wait()
            pltpu.make_async_copy(
                ub1.at[pl.ds(st, CHW)], ubc_v, sub).wait()
            pltpu.make_async_copy(
                ib1.at[pl.ds(st, CHW)], ibc_v, sib).wait()

    sems0_ = (semu0, semi0, semub0, semib0)
    sems1_ = (semu1, semi1, semub1, semib1)

    @pl.when(wid == (NCHF % NW))
    def _():
        pltpu.async_copy(tu, tu_v, semtu)
        pltpu.async_copy(ti, ti_v, semti)
        pltpu.async_copy(tub, tub_v, semtub)
        pltpu.async_copy(tib, tib_v, semtib)

    fire_chunk(wid, 0, sems0_)

    def initb(j, _):
        sl = pl.ds(j * L, L)
        big = jnp.full((L,), 0x7FFFFFF, jnp.int32)
        dmy = jnp.full((L,), DUMMY, jnp.int32)
        uli_v[sl] = big
        ili_v[sl] = big
        ulb_v[sl] = dmy
        ilb_v[sl] = dmy
        return 0
    lax.fori_loop(0, LCAP // L, initb, 0)

    def bucket_pass(idx_hbm, li_v, lb_v):
        pltpu.async_copy(idx_hbm.at[pl.ds(0, 2048)], stage0_v, sems0)
        hcnt = jnp.zeros((L,), jnp.int32)
        for c in range(8):
            buf = c & 1
            sv_cur = stage0_v if buf == 0 else stage1_v
            sv_nxt = stage1_v if buf == 0 else stage0_v
            if c < 7:
                pltpu.async_copy(
                    idx_hbm.at[pl.ds((c + 1) * 2048, 2048)],
                    sv_nxt, (sems1 if buf == 0 else sems0))
            pltpu.make_async_copy(
                idx_hbm.at[pl.ds(c * 2048, 2048)], sv_cur,
                (sems0 if buf == 0 else sems1)).wait()

            def bk(j, h, c=c, sv_cur=sv_cur):
                uvec = sv_cur[pl.ds(j * L, L)]
                m = lax.bitwise_and(
                    lax.shift_right_logical(uvec, 10), NW - 1) == wid
                n = plsc.all_reduce_population_count(m)

                def hit():
                    pos = plsc.cumsum(m.astype(jnp.int32))
                    slots = jnp.minimum(h + pos - 1, VCAP - 1)
                    plsc.store_scatter(li_v, [slots], uvec, mask=m)
                    bvec = c * 2048 + j * L + iota
                    plsc.store_scatter(lb_v, [slots], bvec, mask=m)
                    return h + n
                return lax.cond(n[0] > 0, hit, lambda: h)
            hcnt = lax.fori_loop(0, 2, bk, hcnt)
        return hcnt

    uh = bucket_pass(uidx, uli_v, ulb_v)
    ih = bucket_pass(iidx, ili_v, ilb_v)
    unv = lax.shift_right_logical(
        jnp.minimum(uh[0], VCAP - 1) + (L - 1), 4)
    inv = lax.shift_right_logical(
        jnp.minimum(ih[0], VCAP - 1) + (L - 1), 4)

    def proc_table(k, ch_ref, bc_ref, li_v, nvreg, vv_v, bv_v, tail, t_ref,
                   tb_ref):
        def scanb(j, nh):
            lvec = li_v[pl.ds(j * L, L)]
            m2 = lax.shift_right_logical(lvec, 10) == k
            n = plsc.all_reduce_population_count(m2)

            def hit():
                pos = plsc.cumsum(m2.astype(jnp.int32))
                slots = jnp.minimum(nh + pos - 1, CHCAP - 1)
                plsc.store_scatter(chl_v, [slots], lvec, mask=m2)
                plsc.store_scatter(chs_v, [slots], j * L + iota, mask=m2)
                return nh + n
            return lax.cond(n[0] > 0, hit, lambda: nh)
        nh = lax.fori_loop(0, nvreg, scanb, jnp.zeros((L,), jnp.int32))
        nt = lax.shift_right_logical(nh[0] + (L - 1), 4)

        def dense(t, _):
            hv = chl_v[pl.ds(t * L, L)]
            sv = chs_v[pl.ds(t * L, L)]
            mt = (t * L + iota) < nh
            if tail:
                cols = hv - TAIL0
                for d in range(EMB):
                    vals = plsc.load_gather(t_ref, [cols * EMB + d], mask=mt)
                    plsc.store_scatter(vv_v, [sv * EMB + d], vals, mask=mt)
                bv = plsc.load_gather(tb_ref, [cols], mask=mt)
            else:
                cols = lax.bitwise_and(hv, CHW - 1)
                for d in range(EMB):
                    vals = plsc.load_gather(
                        ch_ref, [jnp.full((L,), d, jnp.int32), cols],
                        mask=mt)
                    plsc.store_scatter(vv_v, [sv * EMB + d], vals, mask=mt)
                bv = plsc.load_gather(bc_ref, [cols], mask=mt)
            plsc.store_scatter(bv_v, [sv], bv, mask=mt)
            return 0
        lax.fori_loop(0, nt, dense, 0)

    def proc_chunk(k, buf, bufsems):
        wait_chunk(k, buf, bufsems)
        return
        ubc_v = ubc0_v if buf == 0 else ubc1_v
        ibc_v = ibc0_v if buf == 0 else ibc1_v
        proc_table(k, uch_v.at[buf], ubc_v, uli_v, unv, uvv_v, ubv_v,
                   False, None, None)
        proc_table(k, ich_v.at[buf], ibc_v, ili_v, inv, ivv_v, ibv_v,
                   False, None, None)

    def pair(p, _):
        ka = wid + p * (2 * NW)
        fire_chunk(ka + NW, 1, sems1_)
        proc_chunk(ka, 0, sems0_)
        fire_chunk(ka + 2 * NW, 0, sems0_)
        proc_chunk(ka + NW, 1, sems1_)
        return 0
    lax.fori_loop(0, 15, pair, 0)

    proc_chunk(wid + 30 * NW, 0, sems0_)

    @pl.when(wid == (NCHF % NW))
    def _():
        pltpu.make_async_copy(tu, tu_v, semtu).wait()
        pltpu.make_async_copy(ti, ti_v, semti).wait()
        pltpu.make_async_copy(tub, tub_v, semtub).wait()
        pltpu.make_async_copy(tib, tib_v, semtib).wait()
        kt = jnp.int32(NCHF)
        proc_table(kt, None, None, uli_v, unv, uvv_v, ubv_v,
                   True, tu_v, tub_v)
        proc_table(kt, None, None, ili_v, inv, ivv_v, ibv_v,
                   True, ti_v, tib_v)

    pltpu.sync_copy(uvv_v, uvals_o.at[pl.ds(wid * VCAP * EMB, VCAP * EMB)])
    pltpu.sync_copy(ivv_v, ivals_o.at[pl.ds(wid * VCAP * EMB, VCAP * EMB)])
    pltpu.sync_copy(ubv_v, ubv_o.at[pl.ds(wid * LCAP, LCAP)])
    pltpu.sync_copy(ibv_v, ibv_o.at[pl.ds(wid * LCAP, LCAP)])
    pltpu.sync_copy(ulb_v, uslot_o.at[pl.ds(wid * LCAP, LCAP)])
    pltpu.sync_copy(ilb_v, islot_o.at[pl.ds(wid * LCAP, LCAP)])


@functools.partial(
    pl.kernel,
    out_type=[
        jax.ShapeDtypeStruct((B + 128, EMB), jnp.float32),
        jax.ShapeDtypeStruct((B + 128, EMB), jnp.float32),
        jax.ShapeDtypeStruct((B + 128,), jnp.float32),
        jax.ShapeDtypeStruct((B + 128,), jnp.float32),
    ],
    mesh=_mesh,
    compiler_params=pltpu.CompilerParams(use_tc_tiling_on_sc=False),
    scratch_types=[
        pltpu.VMEM((VCAP, EMB), jnp.float32),
        pltpu.VMEM((VCAP, EMB), jnp.float32),
        pltpu.VMEM((LCAP // 128, 128), jnp.float32),
        pltpu.VMEM((LCAP // 128, 128), jnp.float32),
        pltpu.VMEM((LCAP // 128, 128), jnp.int32),
        pltpu.VMEM((LCAP // 128, 128), jnp.int32),
        pltpu.SemaphoreType.DMA,
        pltpu.SemaphoreType.DMA,
    ],
)
def _sc_scatter(uvals3, ivals3, ubv3, ibv3, uslot3, islot3,
                ug_o, vg_o, ubg_o, ibg_o,
                uv_v, iv_v, ub_v, ib_v, us_v, is_v, lsem, ssem):
    wid = lax.axis_index("s") * NC + lax.axis_index("c")
    pltpu.async_copy(uvals3.at[wid], uv_v, lsem)
    pltpu.async_copy(ivals3.at[wid], iv_v, lsem)
    pltpu.async_copy(ubv3.at[wid], ub_v, lsem)
    pltpu.async_copy(ibv3.at[wid], ib_v, lsem)
    pltpu.async_copy(uslot3.at[wid], us_v, lsem)
    pltpu.async_copy(islot3.at[wid], is_v, lsem)
    pltpu.make_async_copy(uvals3.at[wid], uv_v, lsem).wait()
    pltpu.make_async_copy(ivals3.at[wid], iv_v, lsem).wait()
    pltpu.make_async_copy(ubv3.at[wid], ub_v, lsem).wait()
    pltpu.make_async_copy(ibv3.at[wid], ib_v, lsem).wait()
    pltpu.make_async_copy(uslot3.at[wid], us_v, lsem).wait()
    pltpu.make_async_copy(islot3.at[wid], is_v, lsem).wait()

    copies = []
    for j in range(VCAP // 128):
        copies.append(pltpu.async_copy(
            uv_v.at[pl.ds(j * 128, 128)], ug_o.at[us_v.at[j]], ssem))
        copies.append(pltpu.async_copy(
            iv_v.at[pl.ds(j * 128, 128)], vg_o.at[is_v.at[j]], ssem))
        copies.append(pltpu.async_copy(
            ub_v.at[j], ubg_o.at[us_v.at[j]], ssem))
        copies.append(pltpu.async_copy(
            ib_v.at[j], ibg_o.at[is_v.at[j]], ssem))
    for c in copies:
        c.wait()


@functools.partial(
    pl.kernel,
    out_type=[
        jax.ShapeDtypeStruct((NW, L), jnp.float32),
        jax.ShapeDtypeStruct((B,), jnp.float32),
    ],
    mesh=_mesh,
    compiler_params=pltpu.CompilerParams(use_tc_tiling_on_sc=False),
    scratch_types=[
        pltpu.VMEM((B // NW, EMB), jnp.float32),
        pltpu.VMEM((B // NW, EMB), jnp.float32),
        pltpu.VMEM((B // NW,), jnp.float32),
        pltpu.VMEM((B // NW,), jnp.float32),
        pltpu.VMEM((B // NW,), jnp.float32),
        pltpu.VMEM((L,), jnp.float32),
        pltpu.SemaphoreType.DMA,
    ],
)
def _sc_dot(ug, vg, ubg, ibg, partials_o, bsum_o,
            u_v, v_v, ub_v, ib_v, bs_v, acc_v, sem):
    wid = lax.axis_index("s") * NC + lax.axis_index("c")
    rpw = B // NW
    base = wid * rpw
    pltpu.async_copy(ug.at[pl.ds(base, rpw)], u_v, sem)
    pltpu.async_copy(vg.at[pl.ds(base, rpw)], v_v, sem)
    pltpu.async_copy(ubg.at[pl.ds(base, rpw)], ub_v, sem)
    pltpu.async_copy(ibg.at[pl.ds(base, rpw)], ib_v, sem)
    pltpu.make_async_copy(ug.at[pl.ds(base, rpw)], u_v, sem).wait()
    pltpu.make_async_copy(vg.at[pl.ds(base, rpw)], v_v, sem).wait()
    pltpu.make_async_copy(ubg.at[pl.ds(base, rpw)], ub_v, sem).wait()
    pltpu.make_async_copy(ibg.at[pl.ds(base, rpw)], ib_v, sem).wait()

    def dot_body(i, acc):
        return acc + u_v[i] * v_v[i]
    acc = lax.fori_loop(0, rpw, dot_body, jnp.zeros((L,), jnp.float32))
    acc_v[...] = acc
    pltpu.sync_copy(acc_v, partials_o.at[wid])

    def bias_body(c, _):
        sl = pl.ds(c * L, L)
        bs_v[sl] = ub_v[sl] + ib_v[sl]
        return 0
    lax.fori_loop(0, rpw // L, bias_body, 0)
    pltpu.sync_copy(bs_v, bsum_o.at[pl.ds(base, rpw)])


def _tc_finalize(partials_ref, bsum_ref, out_ref):
    s = jnp.sum(partials_ref[...])
    out_ref[...] = jax.nn.sigmoid(s + bsum_ref[...])


def kernel(inputs, user_emb, user_bias, item_emb, item_bias):
    uidx = inputs[:, 0]
    iidx = inputs[:, 1]
    ub1 = user_bias.reshape(-1)
    ib1 = item_bias.reshape(-1)
    tu = user_emb[TAIL0:].reshape(-1)
    ti = item_emb[TAIL0:].reshape(-1)
    tub = ub1[TAIL0:]
    tib = ib1[TAIL0:]

    uvals, ivals, ubv, ibv, uslot, islot = _sc_scan(
        user_emb.T, item_emb.T, ub1, ib1, tu, ti, tub, tib, uidx, iidx)

    ug, vg, ubg, ibg = _sc_scatter(
        uvals.reshape(NW, VCAP, EMB), ivals.reshape(NW, VCAP, EMB),
        ubv.reshape(NW, LCAP // 128, 128), ibv.reshape(NW, LCAP // 128, 128),
        uslot.reshape(NW, LCAP // 128, 128),
        islot.reshape(NW, LCAP // 128, 128))

    partials, bsum = _sc_dot(ug[:B], vg[:B], ubg[:B], ibg[:B])

    out = pl.pallas_call(
        _tc_finalize,
        out_shape=jax.ShapeDtypeStruct((B // 128, 128), jnp.float32),
    )(partials, bsum.reshape(B // 128, 128))
    return out.reshape(B, 1)


# speedup vs baseline: 5.0858x; 5.0858x over previous
import functools

import jax
import jax.numpy as jnp
from jax import lax
from jax.experimental import pallas as pl
from jax.experimental.pallas import tpu as pltpu
from jax.experimental.pallas import tpu_sc as plsc

B = 16384
EMB = 16
NC = 2
NS = 16
L = 16
NW = NC * NS
RPW = B // NW
CH = 128
NCH = RPW // CH

_mesh = plsc.VectorSubcoreMesh(core_axis_name="c", subcore_axis_name="s")


@functools.partial(
    pl.kernel,
    out_type=[
        jax.ShapeDtypeStruct((NW, L), jnp.float32),
        jax.ShapeDtypeStruct((B // CH, CH), jnp.float32),
    ],
    mesh=_mesh,
    compiler_params=pltpu.CompilerParams(use_tc_tiling_on_sc=False),
    scratch_types=[
        pltpu.VMEM((NCH, CH), jnp.int32),
        pltpu.VMEM((NCH, CH), jnp.int32),
        pltpu.VMEM((NCH * EMB, CH), jnp.float32),
        pltpu.VMEM((NCH * EMB, CH), jnp.float32),
        pltpu.VMEM((NCH, CH), jnp.float32),
        pltpu.VMEM((NCH, CH), jnp.float32),
        pltpu.VMEM((NCH, CH), jnp.float32),
        pltpu.VMEM((L,), jnp.float32),
        pltpu.SemaphoreType.DMA,
        pltpu.SemaphoreType.DMA,
    ],
)
def _sc_gather_dot(*args):
    (uidx_hbm, iidx_hbm, ubias_hbm, ibias_hbm) = args[:4]
    ud = args[4:4 + EMB]
    idt = args[4 + EMB:4 + 2 * EMB]
    (partials_hbm, bsum_hbm,
     uidx_v, iidx_v, ug_v, ig_v, ub_v, ib_v, bs_v, acc_v,
     sem, bsem) = args[4 + 2 * EMB:]

    wid = lax.axis_index("s") * NC + lax.axis_index("c")
    rbase = wid * NCH

    pltpu.sync_copy(uidx_hbm.at[pl.ds(rbase, NCH)], uidx_v)
    pltpu.sync_copy(iidx_hbm.at[pl.ds(rbase, NCH)], iidx_v)

    copies = []
    for j in range(NCH):
        copies.append(
            pltpu.async_copy(ubias_hbm.at[uidx_v.at[j]], ub_v.at[j], bsem))
        copies.append(
            pltpu.async_copy(ibias_hbm.at[iidx_v.at[j]], ib_v.at[j], bsem))
        for d in range(EMB):
            copies.append(pltpu.async_copy(
                ud[d].at[uidx_v.at[j]], ug_v.at[j * EMB + d], sem))
            copies.append(pltpu.async_copy(
                idt[d].at[iidx_v.at[j]], ig_v.at[j * EMB + d], sem))
    for c in copies:
        c.wait()

    def dot_body(i, acc):
        r = lax.shift_right_logical(i, 3)
        t = lax.bitwise_and(i, 7)
        sl = pl.ds(t * L, L)
        return acc + ug_v[r, sl] * ig_v[r, sl]
    acc = lax.fori_loop(0, NCH * EMB * (CH // L), dot_body,
                        jnp.zeros((L,), jnp.float32))
    acc_v[...] = acc
    pltpu.sync_copy(acc_v, partials_hbm.at[wid])

    for j in range(NCH):
        def bias_body(cidx, _, j=j):
            sl = pl.ds(cidx * L, L)
            bs_v[j, sl] = ub_v[j, sl] + ib_v[j, sl]
            return 0
        lax.fori_loop(0, CH // L, bias_body, 0)
    pltpu.sync_copy(bs_v, bsum_hbm.at[pl.ds(rbase, NCH)])


def _tc_finalize(partials_ref, bsum_ref, out_ref):
    s = jnp.sum(partials_ref[...])
    out_ref[...] = jax.nn.sigmoid(s + bsum_ref[...])


def kernel(inputs, user_emb, user_bias, item_emb, item_bias):
    uidx = inputs[:, 0].reshape(B // CH, CH)
    iidx = inputs[:, 1].reshape(B // CH, CH)
    ub = user_bias.reshape(-1)
    ib = item_bias.reshape(-1)
    uds = [user_emb[:, d] for d in range(EMB)]
    ids_ = [item_emb[:, d] for d in range(EMB)]
    partials, bsum = _sc_gather_dot(uidx, iidx, ub, ib, *uds, *ids_)
    out = pl.pallas_call(
        _tc_finalize,
        out_shape=jax.ShapeDtypeStruct((B // CH, CH), jnp.float32),
    )(partials, bsum)
    return out.reshape(B, 1)
